# baseline (speedup 1.0000x reference)
import functools

import jax
import jax.numpy as jnp
from jax import lax
from jax.experimental import pallas as pl
from jax.experimental.pallas import tpu as pltpu
from jax.experimental.pallas import tpu_sc as plsc

_B = 16384
_H = 50
_D = 32
_V = 1000000

_NW = 32
_TB = _V // 128
_TAIL = _V - _TB * 128
_NBLK = (_TB + _NW - 1) // _NW

_BPW = _B // _NW


def _make_transpose():
  mesh = plsc.VectorSubcoreMesh(core_axis_name="c", subcore_axis_name="s")

  @functools.partial(
      pl.kernel,
      mesh=mesh,
      out_type=jax.ShapeDtypeStruct((250000, 128), jnp.float32),
      scratch_types=[
          pltpu.VMEM((32, 128), jnp.float32),
          pltpu.VMEM((32, 128), jnp.float32),
          pltpu.VMEM((32, 64), jnp.float32),
          pltpu.VMEM((16, 128), jnp.float32),
      ],
      compiler_params=pltpu.CompilerParams(use_tc_tiling_on_sc=True,
                                           needs_layout_passes=False,
                                           disable_bounds_checks=True),
  )
  def tk(tt_hbm, t2_hbm, gbuf, sbuf, gtail, stail):
    wid = lax.axis_index("s") * 2 + lax.axis_index("c")
    iota = lax.iota(jnp.int32, 16)
    rows01 = (iota, iota + 16)

    def blk(jj, carry):
      j = wid + _NW * jj

      @pl.when(j < _TB)
      def _():
        pltpu.sync_copy(tt_hbm.at[:, pl.ds(j * 128, 128)], gbuf)
        for r in range(32):
          for k in range(8):
            vals = plsc.load_gather(
                gbuf, [rows01[k % 2],
                       jnp.full((16,), 4 * r + k // 2, jnp.int32)])
            sbuf[r, pl.ds(16 * k, 16)] = vals
        pltpu.sync_copy(sbuf, t2_hbm.at[pl.ds(j * 32, 32), :])

      return carry

    lax.fori_loop(0, _NBLK, blk, 0)

    @pl.when(wid == 0)
    def _():
      pltpu.sync_copy(tt_hbm.at[:, pl.ds(_TB * 128, _TAIL)], gtail)
      for r in range(16):
        for k in range(8):
          vals = plsc.load_gather(
              gtail, [rows01[k % 2],
                      jnp.full((16,), 4 * r + k // 2, jnp.int32)])
          stail[r, pl.ds(16 * k, 16)] = vals
      pltpu.sync_copy(stail, t2_hbm.at[pl.ds(_TB * 32, 16), :])

  return tk


def _make_gather():
  mesh = plsc.VectorSubcoreMesh(core_axis_name="c", subcore_axis_name="s")

  @functools.partial(
      pl.kernel,
      mesh=mesh,
      out_type=jax.ShapeDtypeStruct((_H, 4, 128, 8, 128), jnp.float32),
      scratch_types=[
          pltpu.VMEM((_BPW,), jnp.int32),
          pltpu.VMEM((_BPW, _D), jnp.float32),
          pltpu.VMEM((4, 4, 8, 128), jnp.float32),
          pltpu.SemaphoreType.DMA,
      ],
      compiler_params=pltpu.CompilerParams(use_tc_tiling_on_sc=False,
                                           needs_layout_passes=False,
                                           disable_bounds_checks=True),
  )
  def gk(t_hbm, idx_hbm, out_hbm, idx_v, rows_v, stg, sem):
    wid = lax.axis_index("s") * 2 + lax.axis_index("c")
    b0 = wid * _BPW
    iota = lax.iota(jnp.int32, 16)

    def hloop(h, carry):
      pltpu.sync_copy(idx_hbm.at[pl.ds(h * _B + b0, _BPW)], idx_v)
      pltpu.async_copy(t_hbm.at[idx_v], rows_v, sem).wait()

      def kloop(k, c2):
        base = 16 * k
        for jp in range(4):
          ridx = 128 * jp + base + iota
          for i in range(4):
            for f in range(8):
              vals = plsc.load_gather(
                  rows_v, [ridx, jnp.full((16,), 8 * i + f, jnp.int32)])
              stg[i, jp, f, pl.ds(base, 16)] = vals
        return c2

      lax.fori_loop(0, 8, kloop, 0)
      for i in range(4):
        pltpu.sync_copy(stg.at[i], out_hbm.at[h, i, pl.ds(4 * wid, 4)])
      return carry

    lax.fori_loop(0, _H, hloop, 0)

  return gk


_transpose = _make_transpose()
_gather = _make_gather()


def kernel(x, table):
  t2 = _transpose(table.T)
  t_lin = t2.reshape(_V, _D)
  idxT = x.T.reshape(_B * _H)
  out5 = _gather(t_lin, idxT)
  return out5.transpose(2, 4, 0, 1, 3).reshape(_B, _H, _D)


# speedup vs baseline: 1.5320x; 1.5320x over previous
import functools

import jax
import jax.numpy as jnp
from jax import lax
from jax.experimental import pallas as pl
from jax.experimental.pallas import tpu as pltpu
from jax.experimental.pallas import tpu_sc as plsc

_B = 16384
_H = 50
_D = 32
_V = 1000000

_NW = 32
_TB = _V // 128
_TAIL = _V - _TB * 128
_NPAIR = (_TB + 2 * _NW - 1) // (2 * _NW)

_BPW = _B // _NW
_CPW = _BPW // 4


def _make_transpose():
  mesh = plsc.VectorSubcoreMesh(core_axis_name="c", subcore_axis_name="s")

  @functools.partial(
      pl.kernel,
      mesh=mesh,
      out_type=jax.ShapeDtypeStruct((_V * _D,), jnp.float32),
      scratch_types=[
          pltpu.VMEM((32, 128), jnp.float32),
          pltpu.VMEM((32, 128), jnp.float32),
          pltpu.VMEM((4096,), jnp.float32),
          pltpu.VMEM((4096,), jnp.float32),
          pltpu.VMEM((32, 64), jnp.float32),
          pltpu.VMEM((2048,), jnp.float32),
          pltpu.SemaphoreType.DMA,
          pltpu.SemaphoreType.DMA,
          pltpu.SemaphoreType.DMA,
          pltpu.SemaphoreType.DMA,
      ],
      compiler_params=pltpu.CompilerParams(use_tc_tiling_on_sc=True,
                                           needs_layout_passes=False,
                                           disable_bounds_checks=True),
  )
  def tk(tt_hbm, t2_hbm, gbuf0, gbuf1, sbuf0, sbuf1, gtail, stail,
         si0, si1, so0, so1):
    gbuf = (gbuf0, gbuf1)
    sbuf = (sbuf0, sbuf1)
    si = (si0, si1)
    so = (so0, so1)
    wid = lax.axis_index("s") * 2 + lax.axis_index("c")
    iota = lax.iota(jnp.int32, 16)
    pat = [((16 * m + iota) // 4) * 128 + ((16 * m + iota) % 4) * 32
           for m in range(8)]

    def in_start(j, s):
      pltpu.async_copy(tt_hbm.at[:, pl.ds(j * 128, 128)], gbuf[s], si[s])

    def in_wait(j, s):
      pltpu.make_async_copy(tt_hbm.at[:, pl.ds(j * 128, 128)], gbuf[s],
                            si[s]).wait()

    def out_start(j, s):
      pltpu.async_copy(sbuf[s], t2_hbm.at[pl.ds(j * 4096, 4096)], so[s])

    def out_wait(s):
      pltpu.make_async_copy(sbuf[s], t2_hbm.at[pl.ds(0, 4096)], so[s]).wait()

    def vec(s):
      g, sb = gbuf[s], sbuf[s]
      for e in range(32):
        vals = [g[e, pl.ds(16 * m, 16)] for m in range(8)]
        for m in range(8):
          plsc.store_scatter(sb, [pat[m] + e], vals[m])

    in_start(wid, 0)
    in_start(wid + _NW, 1)

    def blk2(ii, carry):
      for s in range(2):
        jj = 2 * ii + s
        j = wid + _NW * jj

        @pl.when(j < _TB)
        def _():
          in_wait(j, s)

          @pl.when(jj >= 2)
          def _():
            out_wait(s)

          vec(s)
          out_start(j, s)

          @pl.when(j + 2 * _NW < _TB)
          def _():
            in_start(j + 2 * _NW, s)

      return carry

    lax.fori_loop(0, _NPAIR, blk2, 0)
    out_wait(0)
    out_wait(1)

    @pl.when(wid == 0)
    def _():
      pltpu.sync_copy(tt_hbm.at[:, pl.ds(_TB * 128, _TAIL)], gtail)
      for e in range(32):
        vals = [gtail[e, pl.ds(16 * m, 16)] for m in range(4)]
        for m in range(4):
          plsc.store_scatter(stail, [pat[m] + e], vals[m])
      pltpu.sync_copy(stail, t2_hbm.at[pl.ds(_TB * 4096, 2048)])

  return tk


def _make_gather():
  mesh = plsc.VectorSubcoreMesh(core_axis_name="c", subcore_axis_name="s")

  @functools.partial(
      pl.kernel,
      mesh=mesh,
      out_type=jax.ShapeDtypeStruct((_H, 4, 131072), jnp.float32),
      scratch_types=[
          pltpu.VMEM((_CPW,), jnp.int32),
          pltpu.VMEM((_CPW,), jnp.int32),
          pltpu.VMEM((_CPW, _D), jnp.float32),
          pltpu.VMEM((_CPW, _D), jnp.float32),
          pltpu.VMEM((4096,), jnp.float32),
          pltpu.VMEM((4096,), jnp.float32),
          pltpu.SemaphoreType.DMA,
          pltpu.SemaphoreType.DMA,
          pltpu.SemaphoreType.DMA,
          pltpu.SemaphoreType.DMA,
      ],
      compiler_params=pltpu.CompilerParams(use_tc_tiling_on_sc=False,
                                           needs_layout_passes=False,
                                           disable_bounds_checks=True),
  )
  def gk(t_hbm, idx_hbm, out_hbm, idx0, idx1, rows0, rows1, stg0, stg1,
         sg0, sg1, sn0, sn1):
    idxv = (idx0, idx1)
    rows = (rows0, rows1)
    stg = (stg0, stg1)
    sg = (sg0, sg1)
    so = (sn0, sn1)
    wid = lax.axis_index("s") * 2 + lax.axis_index("c")
    iota = lax.iota(jnp.int32, 16)
    qpat = [((16 * q + iota) // 8) * 1024 + ((16 * q + iota) % 8) * 128
            for q in range(2)]

    def idx_load(jo, s):
      off = (jo // 4) * _B + wid * _BPW + (jo % 4) * _CPW
      pltpu.sync_copy(idx_hbm.at[pl.ds(off, _CPW)], idxv[s])

    def g_start(s):
      pltpu.async_copy(t_hbm.at[idxv[s]], rows[s], sg[s])

    def g_wait(s):
      pltpu.make_async_copy(t_hbm.at[idxv[s]], rows[s], sg[s]).wait()

    def out_start(jo, s):
      h = jo // 4
      qoff = 4096 * wid + (jo % 4) * 1024
      for i in range(4):
        pltpu.async_copy(stg[s].at[pl.ds(i * 1024, 1024)],
                         out_hbm.at[h, i, pl.ds(qoff, 1024)], so[s])

    def out_wait(s):
      for i in range(4):
        pltpu.make_async_copy(stg[s].at[pl.ds(i * 1024, 1024)],
                              out_hbm.at[0, i, pl.ds(0, 1024)], so[s]).wait()

    def vec(s):
      r, sb = rows[s], stg[s]
      for t0 in range(0, _CPW, 4):
        vals = []
        for dt in range(4):
          for q in range(2):
            vals.append(r[t0 + dt, pl.ds(16 * q, 16)])
        vi = 0
        for dt in range(4):
          toff = t0 + dt
          for q in range(2):
            plsc.store_scatter(sb, [qpat[q] + toff], vals[vi])
            vi += 1

    idx_load(0, 0)
    g_start(0)
    njobs = 4 * _H

    def hloop(ii, carry):
      for s in range(2):
        jo = 2 * ii + s
        p = 1 - s
        g_wait(s)

        @pl.when(jo < njobs - 1)
        def _():
          idx_load(jo + 1, p)
          g_start(p)

        @pl.when(jo >= 2)
        def _():
          out_wait(s)

        vec(s)
        out_start(jo, s)
      return carry

    lax.fori_loop(0, njobs // 2, hloop, 0)
    out_wait(0)
    out_wait(1)

  return gk


_transpose = _make_transpose()
_gather = _make_gather()


def kernel(x, table):
  t2 = _transpose(table.T)
  t_lin = t2.reshape(_V, _D)
  idxT = x.T.reshape(_B * _H)
  out7 = _gather(t_lin, idxT)
  out5 = out7.reshape(_H, 4, 128, 8, 128)
  return out5.transpose(2, 4, 0, 1, 3).reshape(_B, _H, _D)


# speedup vs baseline: 1.6351x; 1.0673x over previous
import functools

import jax
import jax.numpy as jnp
from jax import lax
from jax.experimental import pallas as pl
from jax.experimental.pallas import tpu as pltpu
from jax.experimental.pallas import tpu_sc as plsc

_B = 16384
_H = 50
_D = 32
_V = 1000000

_NW = 32
_TB = _V // 256
_TAIL = _V - _TB * 256
_NPAIR = (_TB + 2 * _NW - 1) // (2 * _NW)

_BPW = _B // _NW
_CPW = _BPW // 4


def _make_transpose():
  mesh = plsc.VectorSubcoreMesh(core_axis_name="c", subcore_axis_name="s")

  @functools.partial(
      pl.kernel,
      mesh=mesh,
      out_type=jax.ShapeDtypeStruct((_V * _D,), jnp.float32),
      scratch_types=[
          pltpu.VMEM((32, 256), jnp.float32),
          pltpu.VMEM((32, 256), jnp.float32),
          pltpu.VMEM((8192,), jnp.float32),
          pltpu.VMEM((8192,), jnp.float32),
          pltpu.VMEM((32, 64), jnp.float32),
          pltpu.VMEM((2048,), jnp.float32),
          pltpu.SemaphoreType.DMA,
          pltpu.SemaphoreType.DMA,
          pltpu.SemaphoreType.DMA,
          pltpu.SemaphoreType.DMA,
      ],
      compiler_params=pltpu.CompilerParams(use_tc_tiling_on_sc=True,
                                           needs_layout_passes=False,
                                           disable_bounds_checks=True),
  )
  def tk(tt_hbm, t2_hbm, gbuf0, gbuf1, sbuf0, sbuf1, gtail, stail,
         si0, si1, so0, so1):
    gbuf = (gbuf0, gbuf1)
    sbuf = (sbuf0, sbuf1)
    si = (si0, si1)
    so = (so0, so1)
    wid = lax.axis_index("s") * 2 + lax.axis_index("c")
    iota = lax.iota(jnp.int32, 16)
    pat = [((16 * m + iota) // 4) * 128 + ((16 * m + iota) % 4) * 32
           for m in range(16)]

    def in_start(j, s):
      pltpu.async_copy(tt_hbm.at[:, pl.ds(j * 256, 256)], gbuf[s], si[s])

    def in_wait(j, s):
      pltpu.make_async_copy(tt_hbm.at[:, pl.ds(j * 256, 256)], gbuf[s],
                            si[s]).wait()

    def out_start(j, s):
      pltpu.async_copy(sbuf[s], t2_hbm.at[pl.ds(j * 8192, 8192)], so[s])

    def out_wait(s):
      pltpu.make_async_copy(sbuf[s], t2_hbm.at[pl.ds(0, 8192)], so[s]).wait()

    def vec(s):
      g, sb = gbuf[s], sbuf[s]
      for e in range(32):
        vals = [g[e, pl.ds(16 * m, 16)] for m in range(16)]
        for m in range(16):
          plsc.store_scatter(sb, [pat[m] + e], vals[m])

    in_start(wid, 0)
    in_start(wid + _NW, 1)

    def blk2(ii, carry):
      for s in range(2):
        jj = 2 * ii + s
        j = wid + _NW * jj

        @pl.when(j < _TB)
        def _():
          in_wait(j, s)

          @pl.when(jj >= 2)
          def _():
            out_wait(s)

          vec(s)
          out_start(j, s)

          @pl.when(j + 2 * _NW < _TB)
          def _():
            in_start(j + 2 * _NW, s)

      return carry

    lax.fori_loop(0, _NPAIR, blk2, 0)
    out_wait(0)
    out_wait(1)

    @pl.when(wid == 0)
    def _():
      pltpu.sync_copy(tt_hbm.at[:, pl.ds(_TB * 256, _TAIL)], gtail)
      for e in range(32):
        vals = [gtail[e, pl.ds(16 * m, 16)] for m in range(4)]
        for m in range(4):
          plsc.store_scatter(stail, [pat[m] + e], vals[m])
      pltpu.sync_copy(stail, t2_hbm.at[pl.ds(_TB * 8192, 2048)])

  return tk


def _make_gather():
  mesh = plsc.VectorSubcoreMesh(core_axis_name="c", subcore_axis_name="s")

  @functools.partial(
      pl.kernel,
      mesh=mesh,
      out_type=jax.ShapeDtypeStruct((_H, 4, 131072), jnp.float32),
      scratch_types=[
          pltpu.VMEM((_CPW,), jnp.int32),
          pltpu.VMEM((_CPW,), jnp.int32),
          pltpu.VMEM((_CPW, _D), jnp.float32),
          pltpu.VMEM((_CPW, _D), jnp.float32),
          pltpu.VMEM((4096,), jnp.float32),
          pltpu.VMEM((4096,), jnp.float32),
          pltpu.SemaphoreType.DMA,
          pltpu.SemaphoreType.DMA,
          pltpu.SemaphoreType.DMA,
          pltpu.SemaphoreType.DMA,
          pltpu.SemaphoreType.DMA,
          pltpu.SemaphoreType.DMA,
      ],
      compiler_params=pltpu.CompilerParams(use_tc_tiling_on_sc=False,
                                           needs_layout_passes=False,
                                           disable_bounds_checks=True),
  )
  def gk(t_hbm, idx_hbm, out_hbm, idx0, idx1, rows0, rows1, stg0, stg1,
         sg0, sg1, sn0, sn1, su0, su1):
    su = (su0, su1)
    idxv = (idx0, idx1)
    rows = (rows0, rows1)
    stg = (stg0, stg1)
    sg = (sg0, sg1)
    so = (sn0, sn1)
    wid = lax.axis_index("s") * 2 + lax.axis_index("c")
    iota = lax.iota(jnp.int32, 16)
    qpat = [((16 * q + iota) // 8) * 1024 + ((16 * q + iota) % 8) * 128
            for q in range(2)]

    def idx_start(jo, s):
      off = (jo // 4) * _B + wid * _BPW + (jo % 4) * _CPW
      pltpu.async_copy(idx_hbm.at[pl.ds(off, _CPW)], idxv[s], su[s])

    def idx_wait(s):
      pltpu.make_async_copy(idx_hbm.at[pl.ds(0, _CPW)], idxv[s], su[s]).wait()

    def g_start(s):
      pltpu.async_copy(t_hbm.at[idxv[s]], rows[s], sg[s])

    def g_wait(s):
      pltpu.make_async_copy(t_hbm.at[idxv[s]], rows[s], sg[s]).wait()

    def out_start(jo, s):
      h = jo // 4
      qoff = 4096 * wid + (jo % 4) * 1024
      for i in range(4):
        pltpu.async_copy(stg[s].at[pl.ds(i * 1024, 1024)],
                         out_hbm.at[h, i, pl.ds(qoff, 1024)], so[s])

    def out_wait(s):
      for i in range(4):
        pltpu.make_async_copy(stg[s].at[pl.ds(i * 1024, 1024)],
                              out_hbm.at[0, i, pl.ds(0, 1024)], so[s]).wait()

    def vec(s):
      r, sb = rows[s], stg[s]
      for t0 in range(0, _CPW, 4):
        vals = []
        for dt in range(4):
          for q in range(2):
            vals.append(r[t0 + dt, pl.ds(16 * q, 16)])
        vi = 0
        for dt in range(4):
          toff = t0 + dt
          for q in range(2):
            plsc.store_scatter(sb, [qpat[q] + toff], vals[vi])
            vi += 1

    njobs = 4 * _H
    idx_start(0, 0)
    idx_start(1, 1)
    idx_wait(0)
    g_start(0)

    def hloop(ii, carry):
      for s in range(2):
        jo = 2 * ii + s
        p = 1 - s
        g_wait(s)

        @pl.when(jo + 2 < njobs)
        def _():
          idx_start(jo + 2, s)

        @pl.when(jo < njobs - 1)
        def _():
          idx_wait(p)
          g_start(p)

        @pl.when(jo >= 2)
        def _():
          out_wait(s)

        vec(s)
        out_start(jo, s)
      return carry

    lax.fori_loop(0, njobs // 2, hloop, 0)
    out_wait(0)
    out_wait(1)

  return gk


_transpose = _make_transpose()
_gather = _make_gather()


def kernel(x, table):
  t2 = _transpose(table.T)
  t_lin = t2.reshape(_V, _D)
  idxT = x.T.reshape(_B * _H)
  out7 = _gather(t_lin, idxT)
  out5 = out7.reshape(_H, 4, 128, 8, 128)
  return out5.transpose(2, 4, 0, 1, 3).reshape(_B, _H, _D)
